# baseline (speedup 1.0000x reference)
import jax
import jax.numpy as jnp
from jax.experimental import pallas as pl
from jax.experimental.pallas import tpu as pltpu

_N1 = 2048
_N2 = 2048
_D = 16
_H = 1024


def _acosh(v):
    return jnp.log(v + jnp.sqrt(v * v - 1.0))


def _aug_x(x):
    xn = jnp.sum(x * x, axis=1, keepdims=True)
    c = 2.0 / (1.0 - xn)
    return jnp.concatenate([x * (-2.0 * c), xn * c, c], axis=1)


def _aug_y(y):
    yn = jnp.sum(y * y, axis=1, keepdims=True)
    b = 1.0 / (1.0 - yn)
    return jnp.concatenate([y * b, b, yn * b], axis=1)


def _mm(a, b):
    return jax.lax.dot_general(
        a, b, (((1,), (1,)), ((), ())), preferred_element_type=jnp.float32)


def _hausdorff_kernel(x_hbm, y_hbm, out_ref, x_v, y_v,
                      sy0, sx0, sy1, sx1):
    cy0 = pltpu.make_async_copy(y_hbm.at[pl.ds(0, _H), :], y_v.at[pl.ds(0, _H), :], sy0)
    cx0 = pltpu.make_async_copy(x_hbm.at[pl.ds(0, _H), :], x_v.at[pl.ds(0, _H), :], sx0)
    cy1 = pltpu.make_async_copy(y_hbm.at[pl.ds(_H, _H), :], y_v.at[pl.ds(_H, _H), :], sy1)
    cx1 = pltpu.make_async_copy(x_hbm.at[pl.ds(_H, _H), :], x_v.at[pl.ds(_H, _H), :], sx1)
    cy0.start()
    cx0.start()
    cy1.start()
    cx1.start()

    cy0.wait()
    ay0 = _aug_y(y_v[pl.ds(0, _H), :])
    cx0.wait()
    ax0 = _aug_x(x_v[pl.ds(0, _H), :])

    m00 = _mm(ax0, ay0)
    r0 = jnp.min(m00, axis=1, keepdims=True)
    c0 = jnp.min(m00, axis=0, keepdims=True)

    cy1.wait()
    ay1 = _aug_y(y_v[pl.ds(_H, _H), :])
    m01 = _mm(ax0, ay1)
    r0 = jnp.minimum(r0, jnp.min(m01, axis=1, keepdims=True))
    c1 = jnp.min(m01, axis=0, keepdims=True)

    cx1.wait()
    ax1 = _aug_x(x_v[pl.ds(_H, _H), :])
    m10 = _mm(ax1, ay0)
    r1 = jnp.min(m10, axis=1, keepdims=True)
    c0 = jnp.minimum(c0, jnp.min(m10, axis=0, keepdims=True))

    m11 = _mm(ax1, ay1)
    r1 = jnp.minimum(r1, jnp.min(m11, axis=1, keepdims=True))
    c1 = jnp.minimum(c1, jnp.min(m11, axis=0, keepdims=True))

    rsum = jnp.sum(_acosh(1.0 + r0)) + jnp.sum(_acosh(1.0 + r1))
    csum = jnp.sum(_acosh(1.0 + c0)) + jnp.sum(_acosh(1.0 + c1))
    out_ref[...] = jnp.reshape(rsum / _N1 + csum / _N2, (1, 1))


def kernel(set1, set2):
    out = pl.pallas_call(
        _hausdorff_kernel,
        out_shape=jax.ShapeDtypeStruct((1, 1), jnp.float32),
        in_specs=[
            pl.BlockSpec(memory_space=pl.ANY),
            pl.BlockSpec(memory_space=pl.ANY),
        ],
        out_specs=pl.BlockSpec(memory_space=pltpu.VMEM),
        scratch_shapes=[
            pltpu.VMEM((_N1, _D), jnp.float32),
            pltpu.VMEM((_N2, _D), jnp.float32),
            pltpu.SemaphoreType.DMA,
            pltpu.SemaphoreType.DMA,
            pltpu.SemaphoreType.DMA,
            pltpu.SemaphoreType.DMA,
        ],
    )(set1, set2)
    return out[0, 0]
